# baseline (speedup 1.0000x reference)
import jax
import jax.numpy as jnp
from jax.experimental import pallas as pl
from jax.experimental.pallas import tpu as pltpu


def _add_body(pos_ref, x_ref, table_ref, out_ref):
    out_ref[0] = x_ref[0] + table_ref[...]


def kernel(encoded_patches, position_embedding, positions):
    batch, num_patches, dim = encoded_patches.shape
    blk_d = 256

    grid_spec = pltpu.PrefetchScalarGridSpec(
        num_scalar_prefetch=1,
        grid=(dim // blk_d, batch),
        in_specs=[
            pl.BlockSpec((1, num_patches, blk_d), lambda d, b, pos: (b, pos[0] // num_patches, d)),
            pl.BlockSpec((num_patches, blk_d), lambda d, b, pos: (pos[0] // num_patches, d)),
        ],
        out_specs=pl.BlockSpec((1, num_patches, blk_d), lambda d, b, pos: (b, 0, d)),
    )

    return pl.pallas_call(
        _add_body,
        grid_spec=grid_spec,
        out_shape=jax.ShapeDtypeStruct(encoded_patches.shape, encoded_patches.dtype),
    )(positions, encoded_patches, position_embedding)
